# initial kernel scaffold (unmeasured)
import jax
import jax.numpy as jnp
from jax import lax
from jax.experimental import pallas as pl
from jax.experimental.pallas import tpu as pltpu


def kernel(
    x,
):
    def body(*refs):
        pass

    out_shape = jax.ShapeDtypeStruct(..., jnp.float32)
    return pl.pallas_call(body, out_shape=out_shape)(...)


# baseline (device time: 22659 ns/iter reference)
import jax
import jax.numpy as jnp
from jax import lax
from jax.experimental import pallas as pl
from jax.experimental.pallas import tpu as pltpu


def kernel(x):
    m_per, n = x.shape

    def body(x_ref, out_ref, comm_ref, send_sem, recv_sem):
        my_x = lax.axis_index("x")
        my_y = lax.axis_index("y")
        my_z = lax.axis_index("z")

        comm_ref[...] = x_ref[...].astype(jnp.bfloat16)
        out_ref[pl.ds(my_x * m_per, m_per), :] = comm_ref[...]

        rdma = pltpu.make_async_remote_copy(
            src_ref=comm_ref,
            dst_ref=out_ref.at[pl.ds(my_x * m_per, m_per), :],
            send_sem=send_sem,
            recv_sem=recv_sem,
            device_id=(1 - my_x, my_y, my_z),
            device_id_type=pltpu.DeviceIdType.MESH,
        )
        rdma.start()
        rdma.wait()

    return pl.pallas_call(
        body,
        out_shape=jax.ShapeDtypeStruct((2 * m_per, n), jnp.bfloat16),
        in_specs=[pl.BlockSpec(memory_space=pltpu.VMEM)],
        out_specs=pl.BlockSpec(memory_space=pltpu.VMEM),
        scratch_shapes=[
            pltpu.VMEM((m_per, n), jnp.bfloat16),
            pltpu.SemaphoreType.DMA,
            pltpu.SemaphoreType.DMA,
        ],
    )(x)


# device time: 16070 ns/iter; 1.4100x vs baseline; 1.4100x over previous
import jax
import jax.numpy as jnp
from jax import lax
from jax.experimental import pallas as pl
from jax.experimental.pallas import tpu as pltpu

Q = 256


def kernel(x):
    m_per, n = x.shape

    def body(x_ref, out_ref, comm_ref, send_sems, rq, ra, ry, rz):
        my_x = lax.axis_index("x")
        my_y = lax.axis_index("y")
        my_z = lax.axis_index("z")
        k = 2 * my_y + my_z
        ka = 3 - k
        ky = 2 * (1 - my_y) + my_z
        kz = 2 * my_y + (1 - my_z)
        base_mine = my_x * m_per
        base_rem = (1 - my_x) * m_per

        comm_ref[...] = x_ref[...].astype(jnp.bfloat16)
        out_ref[pl.ds(base_mine, m_per), :] = comm_ref[...]

        barrier = pltpu.get_barrier_semaphore()
        for dev in ((1 - my_x, my_y, my_z),
                    (my_x, 1 - my_y, my_z),
                    (my_x, my_y, 1 - my_z)):
            pl.semaphore_signal(barrier, inc=1, device_id=dev,
                                device_id_type=pltpu.DeviceIdType.MESH)
        pl.semaphore_wait(barrier, 3)

        a1 = pltpu.make_async_remote_copy(
            src_ref=comm_ref.at[pl.ds(k * Q, Q), :],
            dst_ref=out_ref.at[pl.ds(base_mine + k * Q, Q), :],
            send_sem=send_sems.at[0],
            recv_sem=rq,
            device_id=(1 - my_x, my_y, my_z),
            device_id_type=pltpu.DeviceIdType.MESH,
        )
        a2 = pltpu.make_async_remote_copy(
            src_ref=comm_ref.at[pl.ds(ka * Q, Q), :],
            dst_ref=out_ref.at[pl.ds(base_mine + ka * Q, Q), :],
            send_sem=send_sems.at[1],
            recv_sem=ra,
            device_id=(1 - my_x, my_y, my_z),
            device_id_type=pltpu.DeviceIdType.MESH,
        )
        a1.start()
        a2.start()

        a1.wait_recv()
        f1 = pltpu.make_async_remote_copy(
            src_ref=out_ref.at[pl.ds(base_rem + k * Q, Q), :],
            dst_ref=out_ref.at[pl.ds(base_rem + k * Q, Q), :],
            send_sem=send_sems.at[2],
            recv_sem=ry,
            device_id=(my_x, 1 - my_y, my_z),
            device_id_type=pltpu.DeviceIdType.MESH,
        )
        f2 = pltpu.make_async_remote_copy(
            src_ref=out_ref.at[pl.ds(base_rem + k * Q, Q), :],
            dst_ref=out_ref.at[pl.ds(base_rem + k * Q, Q), :],
            send_sem=send_sems.at[3],
            recv_sem=rz,
            device_id=(my_x, my_y, 1 - my_z),
            device_id_type=pltpu.DeviceIdType.MESH,
        )
        f1.start()
        f2.start()

        a2.wait_recv()
        f1.wait_recv()
        f2.wait_recv()
        a1.wait_send()
        a2.wait_send()
        f1.wait_send()
        f2.wait_send()

    return pl.pallas_call(
        body,
        out_shape=jax.ShapeDtypeStruct((2 * m_per, n), jnp.bfloat16),
        in_specs=[pl.BlockSpec(memory_space=pltpu.VMEM)],
        out_specs=pl.BlockSpec(memory_space=pltpu.VMEM),
        scratch_shapes=[
            pltpu.VMEM((m_per, n), jnp.bfloat16),
            pltpu.SemaphoreType.DMA((4,)),
            pltpu.SemaphoreType.DMA,
            pltpu.SemaphoreType.DMA,
            pltpu.SemaphoreType.DMA,
            pltpu.SemaphoreType.DMA,
        ],
        compiler_params=pltpu.CompilerParams(collective_id=0),
    )(x)


# device time: 13473 ns/iter; 1.6818x vs baseline; 1.1928x over previous
import jax
import jax.numpy as jnp
from jax import lax
from jax.experimental import pallas as pl
from jax.experimental.pallas import tpu as pltpu

Q = 256
H = 128


def kernel(x):
    m_per, n = x.shape

    def body(x_ref, out_ref, send_sems, recv_sems):
        my_x = lax.axis_index("x")
        my_y = lax.axis_index("y")
        my_z = lax.axis_index("z")
        k = 2 * my_y + my_z
        ka = 3 - k
        base_mine = my_x * m_per
        base_rem = (1 - my_x) * m_per
        xp = (1 - my_x, my_y, my_z)
        yp = (my_x, 1 - my_y, my_z)
        zp = (my_x, my_y, 1 - my_z)

        barrier = pltpu.get_barrier_semaphore()
        for dev in (xp, yp, zp):
            pl.semaphore_signal(barrier, inc=1, device_id=dev,
                                device_id_type=pltpu.DeviceIdType.MESH)

        out_ref[pl.ds(base_mine, m_per), :] = x_ref[...].astype(jnp.bfloat16)

        pl.semaphore_wait(barrier, 3)

        def xcopy(rows, nrows, ssem, rsem, dev):
            return pltpu.make_async_remote_copy(
                src_ref=out_ref.at[pl.ds(rows, nrows), :],
                dst_ref=out_ref.at[pl.ds(rows, nrows), :],
                send_sem=send_sems.at[ssem],
                recv_sem=recv_sems.at[rsem],
                device_id=dev,
                device_id_type=pltpu.DeviceIdType.MESH,
            )

        a1 = xcopy(base_mine + k * Q, H, 0, 0, xp)
        a2 = xcopy(base_mine + k * Q + H, H, 1, 1, xp)
        a3 = xcopy(base_mine + ka * Q, Q, 2, 2, xp)
        a1.start()
        a2.start()
        a3.start()

        a1.wait_recv()
        fy1 = xcopy(base_rem + k * Q, H, 3, 3, yp)
        fz1 = xcopy(base_rem + k * Q, H, 4, 4, zp)
        fy1.start()
        fz1.start()
        a2.wait_recv()
        fy2 = xcopy(base_rem + k * Q + H, H, 5, 5, yp)
        fz2 = xcopy(base_rem + k * Q + H, H, 6, 6, zp)
        fy2.start()
        fz2.start()

        a3.wait_recv()
        fy1.wait_recv()
        fz1.wait_recv()
        fy2.wait_recv()
        fz2.wait_recv()
        a1.wait_send()
        a2.wait_send()
        a3.wait_send()
        fy1.wait_send()
        fz1.wait_send()
        fy2.wait_send()
        fz2.wait_send()

    return pl.pallas_call(
        body,
        out_shape=jax.ShapeDtypeStruct((2 * m_per, n), jnp.bfloat16),
        in_specs=[pl.BlockSpec(memory_space=pltpu.VMEM)],
        out_specs=pl.BlockSpec(memory_space=pltpu.VMEM),
        scratch_shapes=[
            pltpu.SemaphoreType.DMA((7,)),
            pltpu.SemaphoreType.DMA((7,)),
        ],
        compiler_params=pltpu.CompilerParams(collective_id=0),
    )(x)
